# baseline (speedup 1.0000x reference)
import functools

import jax
import jax.numpy as jnp
from jax import lax
from jax.experimental import pallas as pl
from jax.experimental.pallas import tpu as pltpu
from jax.experimental.pallas import tpu_sc as plsc

N_NODES = 10000
N_EDGES = 320000
HIDDEN = 128
NHEADS = 4
HEAD_DIM = HIDDEN // NHEADS
TEMP = HIDDEN ** (-0.5)
NEG = -3.0e38

NW = 32
E_PER_W = N_EDGES // NW

CH_E = 2000
N_CH = E_PER_W // CH_E
VPC = CH_E // 16

N_PAD = 10240
DEN_PAD = NHEADS * N_PAD
DEN_PER_TILE = DEN_PAD // 16

SUP = 2000
N_SUP = E_PER_W // SUP
CB = 80
N_CB = SUP // CB
GPC = CB // 16
OUT_RPT = N_PAD // 16

BK = 8192
NBLK = -(-N_EDGES // BK)


def _sel4():
  return (lax.broadcasted_iota(jnp.int32, (NHEADS, HIDDEN), 1) // HEAD_DIM ==
          lax.broadcasted_iota(jnp.int32, (NHEADS, HIDDEN), 0)
          ).astype(jnp.float32)


def _k1_body(ei_ref, k_ref, q_ref, dst_o, s0, s1, s2, s3, gm_ref):
  i = pl.program_id(0)
  kq = k_ref[...] * q_ref[...]
  x = jax.lax.dot_general(kq, _sel4(),
                          (((1,), (1,)), ((), ())),
                          preferred_element_type=jnp.float32)
  x = x * TEMP
  s = jnp.where(x >= 0, x, 0.2 * x)
  st = jnp.transpose(s, (1, 0))
  s0[...] = st[0]
  s1[...] = st[1]
  s2[...] = st[2]
  s3[...] = st[3]
  dst_o[...] = ei_ref[1]
  valid = (lax.broadcasted_iota(jnp.int32, (NHEADS, BK), 1) + i * BK
           < N_EDGES)
  st = jnp.where(valid, st, NEG)
  m = jnp.max(st, axis=1, keepdims=True)
  mb = jnp.concatenate(
      [jnp.broadcast_to(m, (NHEADS, 128)),
       jnp.full((8 - NHEADS, 128), NEG, jnp.float32)], axis=0)

  @pl.when(i == 0)
  def _():
    gm_ref[...] = jnp.full((8, 128), NEG, jnp.float32)

  gm_ref[...] = jnp.maximum(gm_ref[...], mb)


def _k1(edge_index, keys, queries):
  es = jax.ShapeDtypeStruct((N_EDGES,), jnp.float32)
  return pl.pallas_call(
      _k1_body,
      grid=(NBLK,),
      in_specs=[pl.BlockSpec((2, BK), lambda i: (0, i)),
                pl.BlockSpec((BK, HIDDEN), lambda i: (i, 0)),
                pl.BlockSpec((BK, HIDDEN), lambda i: (i, 0))],
      out_specs=[pl.BlockSpec((BK,), lambda i: (i,)),
                 pl.BlockSpec((BK,), lambda i: (i,)),
                 pl.BlockSpec((BK,), lambda i: (i,)),
                 pl.BlockSpec((BK,), lambda i: (i,)),
                 pl.BlockSpec((BK,), lambda i: (i,)),
                 pl.BlockSpec((8, 128), lambda i: (0, 0))],
      out_shape=[jax.ShapeDtypeStruct((N_EDGES,), jnp.int32),
                 es, es, es, es,
                 jax.ShapeDtypeStruct((8, 128), jnp.float32)],
  )(edge_index, keys, queries)


def _k235_body(s0, s1, s2, s3, dst_hbm, gm_hbm, v_hbm,
               dpart_hbm, opart_hbm,
               dst_v, w0_v, w1_v, w2_v, w3_v, gm_v, idxd_v,
               idx_a, idx_b, idx_c, v_a, v_b, v_c,
               in_a, in_b, in_c, sc_a, sc_b, sc_c,
               den_sh, out_sh):
  c = lax.axis_index("c")
  s = lax.axis_index("s")
  wid = c * 16 + s
  wheads = (w0_v, w1_v, w2_v, w3_v)
  sheads = (s0, s1, s2, s3)

  def zw(j, _):
    w0_v[pl.ds(j * 16, 16)] = jnp.zeros((16,), jnp.float32)
    return 0
  lax.fori_loop(0, SUP // 16, zw, 0)
  pltpu.sync_copy(w0_v.at[pl.ds(0, 1280)],
                  den_sh.at[pl.ds(s * DEN_PER_TILE, 1280)])
  pltpu.sync_copy(w0_v.at[pl.ds(0, 1280)],
                  den_sh.at[pl.ds(s * DEN_PER_TILE + 1280, 1280)])

  def zv(r, _):
    for cc in range(HIDDEN // 16):
      v_a[r, pl.ds(cc * 16, 16)] = jnp.zeros((16,), jnp.float32)
    return 0
  lax.fori_loop(0, CB, zv, 0)
  for zi in range(OUT_RPT // CB):
    pltpu.sync_copy(v_a, out_sh.at[pl.ds(s * OUT_RPT + zi * CB, CB)])
  pltpu.sync_copy(gm_hbm.at[pl.ds(0, 512)], gm_v)
  plsc.subcore_barrier()

  def _mul(v_ref, idx_ref, sbase):
    for j in range(CB // 16):
      idx_ref[pl.ds(j * 16, 16)] = dst_v[pl.ds(sbase + j * 16, 16)]

    def group(g, _):
      wv = [wheads[h][pl.ds(sbase + g * 16, 16)] for h in range(NHEADS)]
      for f in range(16):
        row = g * 16 + f
        for h in range(NHEADS):
          spl = jnp.broadcast_to(wv[h][f:f + 1], (16,))
          for j2 in range(2):
            col = h * 2 * 16 + j2 * 16
            v_ref[row, pl.ds(col, 16)] = v_ref[row, pl.ds(col, 16)] * spl
      return 0
    lax.fori_loop(0, GPC, group, 0)

  def sup_chunk(si, _):
    base_e = pl.multiple_of(wid * E_PER_W + si * SUP, 8)
    pltpu.sync_copy(dst_hbm.at[pl.ds(base_e, SUP)], dst_v)
    for h in range(NHEADS):
      wh = wheads[h]
      pltpu.sync_copy(sheads[h].at[pl.ds(base_e, SUP)], wh)
      gh = gm_v[pl.ds(h * 128, 16)]

      def vbody(j, _):
        off = j * 16
        wh[pl.ds(off, 16)] = jnp.exp(wh[pl.ds(off, 16)] - gh)
        dv = dst_v[pl.ds(off, 16)]
        idxd_v[pl.ds(off, 16)] = dv + h * N_PAD
        return 0
      lax.fori_loop(0, VPC, vbody, 0)
      pltpu.sync_copy(wh, den_sh.at[idxd_v], add=True)

    bufs = ((v_a, idx_a, in_a, sc_a),
            (v_b, idx_b, in_b, sc_b),
            (v_c, idx_c, in_c, sc_c))

    def fill(buf, sub):
      pltpu.async_copy(v_hbm.at[pl.ds(base_e + sub * CB, CB)],
                       buf[0], buf[2])

    def wait_fill(buf):
      pltpu.make_async_copy(v_hbm.at[pl.ds(base_e, CB)],
                            buf[0], buf[2]).wait()

    def scat(buf):
      pltpu.async_copy(buf[0], out_sh.at[buf[1]], buf[3], add=True)

    def wait_scat(buf):
      pltpu.make_async_copy(buf[0], out_sh.at[buf[1]], buf[3]).wait()

    for k in range(3):
      fill(bufs[k], k)

    def triple(t, _):
      for k in range(3):
        wait_fill(bufs[k])
        _mul(bufs[k][0], bufs[k][1], (3 * t + k) * CB)
        scat(bufs[k])
      for k in range(3):
        wait_scat(bufs[k])
        fill(bufs[k], 3 * t + k + 3)
      return 0
    lax.fori_loop(0, (N_CB - 4) // 3, triple, 0)
    wait_fill(bufs[0])
    _mul(bufs[0][0], bufs[0][1], (N_CB - 4) * CB)
    scat(bufs[0])
    wait_scat(bufs[0])
    fill(bufs[0], N_CB - 1)
    for k in (1, 2):
      wait_fill(bufs[k])
      _mul(bufs[k][0], bufs[k][1], (N_CB - 4 + k) * CB)
      scat(bufs[k])
      wait_scat(bufs[k])
    wait_fill(bufs[0])
    _mul(bufs[0][0], bufs[0][1], (N_CB - 1) * CB)
    scat(bufs[0])
    wait_scat(bufs[0])
    return 0
  lax.fori_loop(0, N_SUP, sup_chunk, 0)

  plsc.subcore_barrier()
  pltpu.sync_copy(den_sh.at[pl.ds(s * DEN_PER_TILE, DEN_PER_TILE)],
                  dpart_hbm.at[c].at[pl.ds(s * DEN_PER_TILE, DEN_PER_TILE)])
  pltpu.sync_copy(out_sh.at[pl.ds(s * OUT_RPT, OUT_RPT)],
                  opart_hbm.at[c].at[pl.ds(s * OUT_RPT, OUT_RPT)])


def _k235(s0, s1, s2, s3, dst, gmaxflat, values):
  mesh = plsc.VectorSubcoreMesh(core_axis_name="c", subcore_axis_name="s")
  return pl.kernel(
      _k235_body,
      out_type=[jax.ShapeDtypeStruct((2, DEN_PAD), jnp.float32),
                jax.ShapeDtypeStruct((2, N_PAD, HIDDEN), jnp.float32)],
      mesh=mesh,
      scratch_types=[pltpu.VMEM((SUP,), jnp.int32),
                     pltpu.VMEM((SUP,), jnp.float32),
                     pltpu.VMEM((SUP,), jnp.float32),
                     pltpu.VMEM((SUP,), jnp.float32),
                     pltpu.VMEM((SUP,), jnp.float32),
                     pltpu.VMEM((512,), jnp.float32),
                     pltpu.VMEM((SUP,), jnp.int32),
                     pltpu.VMEM((CB,), jnp.int32),
                     pltpu.VMEM((CB,), jnp.int32),
                     pltpu.VMEM((CB,), jnp.int32),
                     pltpu.VMEM((CB, HIDDEN), jnp.float32),
                     pltpu.VMEM((CB, HIDDEN), jnp.float32),
                     pltpu.VMEM((CB, HIDDEN), jnp.float32),
                     pltpu.SemaphoreType.DMA,
                     pltpu.SemaphoreType.DMA,
                     pltpu.SemaphoreType.DMA,
                     pltpu.SemaphoreType.DMA,
                     pltpu.SemaphoreType.DMA,
                     pltpu.SemaphoreType.DMA,
                     pltpu.VMEM_SHARED((DEN_PAD,), jnp.float32),
                     pltpu.VMEM_SHARED((N_PAD, HIDDEN), jnp.float32)],
  )(s0, s1, s2, s3, dst, gmaxflat, values)


BN = 2048


def _k6_body(p_ref, d_ref, o_ref):
  dsum = d_ref[0:NHEADS, :] + d_ref[NHEADS:2 * NHEADS, :]
  dexp = jax.lax.dot_general(dsum, _sel4(),
                             (((0,), (0,)), ((), ())),
                             preferred_element_type=jnp.float32)
  o = p_ref[0] + p_ref[1]
  o_ref[...] = jnp.where(dexp > 0, o / dexp, 0.0)


def _k6(opart, dpart8):
  return pl.pallas_call(
      _k6_body,
      grid=(pl.cdiv(N_NODES, BN),),
      in_specs=[pl.BlockSpec((2, BN, HIDDEN), lambda i: (0, i, 0)),
                pl.BlockSpec((2 * NHEADS, BN), lambda i: (0, i))],
      out_specs=pl.BlockSpec((BN, HIDDEN), lambda i: (i, 0)),
      out_shape=jax.ShapeDtypeStruct((N_NODES, HIDDEN), jnp.float32),
  )(opart, dpart8)


@jax.jit
def kernel(edge_index, keys, queries, values):
  dst, s0, s1, s2, s3, gmax8 = _k1(edge_index, keys, queries)
  dpart, opart = _k235(s0, s1, s2, s3, dst, gmax8.reshape(-1), values)
  return _k6(opart, dpart.reshape(2 * NHEADS, N_PAD))


# speedup vs baseline: 1.0239x; 1.0239x over previous
import functools

import jax
import jax.numpy as jnp
from jax import lax
from jax.experimental import pallas as pl
from jax.experimental.pallas import tpu as pltpu
from jax.experimental.pallas import tpu_sc as plsc

N_NODES = 10000
N_EDGES = 320000
HIDDEN = 128
NHEADS = 4
HEAD_DIM = HIDDEN // NHEADS
TEMP = HIDDEN ** (-0.5)
NEG = -3.0e38

NW = 32
E_PER_W = N_EDGES // NW

CH_E = 2000
N_CH = E_PER_W // CH_E
VPC = CH_E // 16

N_PAD = 10240
DEN_PAD = NHEADS * N_PAD
DEN_PER_TILE = DEN_PAD // 16

SUP = 2000
N_SUP = E_PER_W // SUP
CB = 80
N_CB = SUP // CB
GPC = CB // 16
OUT_RPT = N_PAD // 16

BK = 16384
NBLK = -(-N_EDGES // BK)


def _sel4():
  return (lax.broadcasted_iota(jnp.int32, (NHEADS, HIDDEN), 1) // HEAD_DIM ==
          lax.broadcasted_iota(jnp.int32, (NHEADS, HIDDEN), 0)
          ).astype(jnp.float32)


def _k1_body(ei_ref, k_ref, q_ref, dst_o, s0, s1, s2, s3, gm_ref):
  i = pl.program_id(0)
  kq = k_ref[...] * q_ref[...]
  x = jax.lax.dot_general(kq, _sel4(),
                          (((1,), (1,)), ((), ())),
                          preferred_element_type=jnp.float32)
  x = x * TEMP
  s = jnp.where(x >= 0, x, 0.2 * x)
  st = jnp.transpose(s, (1, 0))
  s0[...] = st[0]
  s1[...] = st[1]
  s2[...] = st[2]
  s3[...] = st[3]
  dst_o[...] = ei_ref[1]
  valid = (lax.broadcasted_iota(jnp.int32, (NHEADS, BK), 1) + i * BK
           < N_EDGES)
  st = jnp.where(valid, st, NEG)
  m = jnp.max(st, axis=1, keepdims=True)
  mb = jnp.concatenate(
      [jnp.broadcast_to(m, (NHEADS, 128)),
       jnp.full((8 - NHEADS, 128), NEG, jnp.float32)], axis=0)

  @pl.when(i == 0)
  def _():
    gm_ref[...] = jnp.full((8, 128), NEG, jnp.float32)

  gm_ref[...] = jnp.maximum(gm_ref[...], mb)


def _k1(edge_index, keys, queries):
  es = jax.ShapeDtypeStruct((N_EDGES,), jnp.float32)
  return pl.pallas_call(
      _k1_body,
      grid=(NBLK,),
      in_specs=[pl.BlockSpec((2, BK), lambda i: (0, i)),
                pl.BlockSpec((BK, HIDDEN), lambda i: (i, 0)),
                pl.BlockSpec((BK, HIDDEN), lambda i: (i, 0))],
      out_specs=[pl.BlockSpec((BK,), lambda i: (i,)),
                 pl.BlockSpec((BK,), lambda i: (i,)),
                 pl.BlockSpec((BK,), lambda i: (i,)),
                 pl.BlockSpec((BK,), lambda i: (i,)),
                 pl.BlockSpec((BK,), lambda i: (i,)),
                 pl.BlockSpec((8, 128), lambda i: (0, 0))],
      out_shape=[jax.ShapeDtypeStruct((N_EDGES,), jnp.int32),
                 es, es, es, es,
                 jax.ShapeDtypeStruct((8, 128), jnp.float32)],
  )(edge_index, keys, queries)


def _k235_body(s0, s1, s2, s3, dst_hbm, gm_hbm, v_hbm,
               dpart_hbm, opart_hbm,
               dst_v, w0_v, w1_v, w2_v, w3_v, sc_v, gm_v, idxd_v,
               idx_a, idx_b, v_a, v_b,
               in_a, in_b, sc_a, sc_b,
               den_sh, out_sh):
  c = lax.axis_index("c")
  s = lax.axis_index("s")
  wid = c * 16 + s
  wheads = (w0_v, w1_v, w2_v, w3_v)
  sheads = (s0, s1, s2, s3)

  def zw(j, _):
    w0_v[pl.ds(j * 16, 16)] = jnp.zeros((16,), jnp.float32)
    return 0
  lax.fori_loop(0, SUP // 16, zw, 0)
  pltpu.sync_copy(w0_v.at[pl.ds(0, 1280)],
                  den_sh.at[pl.ds(s * DEN_PER_TILE, 1280)])
  pltpu.sync_copy(w0_v.at[pl.ds(0, 1280)],
                  den_sh.at[pl.ds(s * DEN_PER_TILE + 1280, 1280)])

  def zv(r, _):
    for cc in range(HIDDEN // 16):
      v_a[r, pl.ds(cc * 16, 16)] = jnp.zeros((16,), jnp.float32)
    return 0
  lax.fori_loop(0, CB, zv, 0)
  for zi in range(OUT_RPT // CB):
    pltpu.sync_copy(v_a, out_sh.at[pl.ds(s * OUT_RPT + zi * CB, CB)])
  pltpu.sync_copy(gm_hbm.at[pl.ds(0, 512)], gm_v)
  plsc.subcore_barrier()

  def _mul(v_ref, idx_ref, sbase):
    for j in range(CB // 16):
      idx_ref[pl.ds(j * 16, 16)] = dst_v[pl.ds(sbase + j * 16, 16)]

    def group(g, _):
      wv = [wheads[h][pl.ds(sbase + g * 16, 16)] for h in range(NHEADS)]
      for f in range(16):
        row = g * 16 + f
        for h in range(NHEADS):
          spl = jnp.broadcast_to(wv[h][f:f + 1], (16,))
          for j2 in range(2):
            col = h * 2 * 16 + j2 * 16
            v_ref[row, pl.ds(col, 16)] = v_ref[row, pl.ds(col, 16)] * spl
      return 0
    lax.fori_loop(0, GPC, group, 0)

  def sup_chunk(si, _):
    base_e = pl.multiple_of(wid * E_PER_W + si * SUP, 8)
    pltpu.sync_copy(dst_hbm.at[pl.ds(base_e, SUP)], dst_v)
    for h in range(NHEADS):
      pltpu.sync_copy(sheads[h].at[pl.ds(base_e, SUP)], sc_v)
      gh = gm_v[pl.ds(h * 128, 16)]
      wh = wheads[h]

      def vbody(j, _):
        off = j * 16
        sv = sc_v[pl.ds(off, 16)]
        wh[pl.ds(off, 16)] = jnp.exp(sv - gh)
        dv = dst_v[pl.ds(off, 16)]
        idxd_v[pl.ds(off, 16)] = dv + h * N_PAD
        return 0
      lax.fori_loop(0, VPC, vbody, 0)
      pltpu.sync_copy(wh, den_sh.at[idxd_v], add=True)

    pltpu.async_copy(v_hbm.at[pl.ds(base_e, CB)], v_a, in_a)
    pltpu.async_copy(v_hbm.at[pl.ds(base_e + CB, CB)], v_b, in_b)

    def pair(p, _):
      sub_a = 2 * p
      sub_b = 2 * p + 1
      pltpu.make_async_copy(v_hbm.at[pl.ds(base_e, CB)], v_a, in_a).wait()
      _mul(v_a, idx_a, sub_a * CB)
      pltpu.async_copy(v_a, out_sh.at[idx_a], sc_a, add=True)
      pltpu.make_async_copy(v_hbm.at[pl.ds(base_e, CB)], v_b, in_b).wait()
      _mul(v_b, idx_b, sub_b * CB)
      pltpu.async_copy(v_b, out_sh.at[idx_b], sc_b, add=True)
      pltpu.make_async_copy(v_a, out_sh.at[idx_a], sc_a).wait()
      pltpu.async_copy(v_hbm.at[pl.ds(base_e + (sub_a + 2) * CB, CB)],
                       v_a, in_a)
      pltpu.make_async_copy(v_b, out_sh.at[idx_b], sc_b).wait()

      @pl.when(p < N_CB // 2 - 1)
      def _():
        pltpu.async_copy(v_hbm.at[pl.ds(base_e + (sub_b + 2) * CB, CB)],
                         v_b, in_b)
      return 0
    lax.fori_loop(0, N_CB // 2, pair, 0)
    pltpu.make_async_copy(v_hbm.at[pl.ds(base_e, CB)], v_a, in_a).wait()
    _mul(v_a, idx_a, (N_CB - 1) * CB)
    pltpu.async_copy(v_a, out_sh.at[idx_a], sc_a, add=True)
    pltpu.make_async_copy(v_a, out_sh.at[idx_a], sc_a).wait()
    return 0
  lax.fori_loop(0, N_SUP, sup_chunk, 0)

  plsc.subcore_barrier()
  drow = c * NHEADS + s // NHEADS
  dcol = pl.multiple_of((s % NHEADS) * DEN_PER_TILE, 8)
  pltpu.sync_copy(den_sh.at[pl.ds(s * DEN_PER_TILE, DEN_PER_TILE)],
                  dpart_hbm.at[drow].at[pl.ds(dcol, DEN_PER_TILE)])
  pltpu.sync_copy(out_sh.at[pl.ds(s * OUT_RPT, OUT_RPT)],
                  opart_hbm.at[c].at[pl.ds(s * OUT_RPT, OUT_RPT)])


def _k235(s0, s1, s2, s3, dst, gmaxflat, values):
  mesh = plsc.VectorSubcoreMesh(core_axis_name="c", subcore_axis_name="s")
  return pl.kernel(
      _k235_body,
      out_type=[jax.ShapeDtypeStruct((2 * NHEADS, N_PAD), jnp.float32),
                jax.ShapeDtypeStruct((2, N_PAD, HIDDEN), jnp.float32)],
      mesh=mesh,
      scratch_types=[pltpu.VMEM((SUP,), jnp.int32),
                     pltpu.VMEM((SUP,), jnp.float32),
                     pltpu.VMEM((SUP,), jnp.float32),
                     pltpu.VMEM((SUP,), jnp.float32),
                     pltpu.VMEM((SUP,), jnp.float32),
                     pltpu.VMEM((SUP,), jnp.float32),
                     pltpu.VMEM((512,), jnp.float32),
                     pltpu.VMEM((SUP,), jnp.int32),
                     pltpu.VMEM((CB,), jnp.int32),
                     pltpu.VMEM((CB,), jnp.int32),
                     pltpu.VMEM((CB, HIDDEN), jnp.float32),
                     pltpu.VMEM((CB, HIDDEN), jnp.float32),
                     pltpu.SemaphoreType.DMA,
                     pltpu.SemaphoreType.DMA,
                     pltpu.SemaphoreType.DMA,
                     pltpu.SemaphoreType.DMA,
                     pltpu.VMEM_SHARED((DEN_PAD,), jnp.float32),
                     pltpu.VMEM_SHARED((N_PAD, HIDDEN), jnp.float32)],
  )(s0, s1, s2, s3, dst, gmaxflat, values)


BN = 2048


def _k6_body(p_ref, d_ref, o_ref):
  dsum = d_ref[0:NHEADS, :] + d_ref[NHEADS:2 * NHEADS, :]
  dexp = jax.lax.dot_general(dsum, _sel4(),
                             (((0,), (0,)), ((), ())),
                             preferred_element_type=jnp.float32)
  o = p_ref[0] + p_ref[1]
  o_ref[...] = jnp.where(dexp > 0, o / dexp, 0.0)


def _k6(opart, dpart8):
  return pl.pallas_call(
      _k6_body,
      grid=(pl.cdiv(N_NODES, BN),),
      in_specs=[pl.BlockSpec((2, BN, HIDDEN), lambda i: (0, i, 0)),
                pl.BlockSpec((2 * NHEADS, BN), lambda i: (0, i))],
      out_specs=pl.BlockSpec((BN, HIDDEN), lambda i: (i, 0)),
      out_shape=jax.ShapeDtypeStruct((N_NODES, HIDDEN), jnp.float32),
  )(opart, dpart8)


@jax.jit
def kernel(edge_index, keys, queries, values):
  dst, s0, s1, s2, s3, gmax8 = _k1(edge_index, keys, queries)
  dpart, opart = _k235(s0, s1, s2, s3, dst, gmax8.reshape(-1), values)
  return _k6(opart, dpart)


# speedup vs baseline: 1.0369x; 1.0127x over previous
import functools

import jax
import jax.numpy as jnp
from jax import lax
from jax.experimental import pallas as pl
from jax.experimental.pallas import tpu as pltpu
from jax.experimental.pallas import tpu_sc as plsc

N_NODES = 10000
N_EDGES = 320000
HIDDEN = 128
NHEADS = 4
HEAD_DIM = HIDDEN // NHEADS
TEMP = HIDDEN ** (-0.5)
NEG = -3.0e38

NW = 32
E_PER_W = N_EDGES // NW

CH_E = 2000
N_CH = E_PER_W // CH_E
VPC = CH_E // 16

N_PAD = 10240
DEN_PAD = NHEADS * N_PAD
DEN_PER_TILE = DEN_PAD // 16

SUP = 2000
N_SUP = E_PER_W // SUP
CB = 80
N_CB = SUP // CB
GPC = CB // 16
OUT_RPT = N_PAD // 16

BK = 16384
NBLK = -(-N_EDGES // BK)


def _sel4():
  return (lax.broadcasted_iota(jnp.int32, (NHEADS, HIDDEN), 1) // HEAD_DIM ==
          lax.broadcasted_iota(jnp.int32, (NHEADS, HIDDEN), 0)
          ).astype(jnp.float32)


def _k1_body(ei_ref, k_ref, q_ref, dst_o, s0, s1, s2, s3, gm_ref):
  i = pl.program_id(0)
  kq = k_ref[...] * q_ref[...]
  x = jax.lax.dot_general(kq, _sel4(),
                          (((1,), (1,)), ((), ())),
                          preferred_element_type=jnp.float32)
  x = x * TEMP
  s = jnp.where(x >= 0, x, 0.2 * x)
  st = jnp.transpose(s, (1, 0))
  s0[...] = st[0]
  s1[...] = st[1]
  s2[...] = st[2]
  s3[...] = st[3]
  dst_o[...] = ei_ref[1]
  valid = (lax.broadcasted_iota(jnp.int32, (NHEADS, BK), 1) + i * BK
           < N_EDGES)
  st = jnp.where(valid, st, NEG)
  m = jnp.max(st, axis=1, keepdims=True)
  mb = jnp.concatenate(
      [jnp.broadcast_to(m, (NHEADS, 128)),
       jnp.full((8 - NHEADS, 128), NEG, jnp.float32)], axis=0)

  @pl.when(i == 0)
  def _():
    gm_ref[...] = jnp.full((8, 128), NEG, jnp.float32)

  gm_ref[...] = jnp.maximum(gm_ref[...], mb)


def _k1(edge_index, keys, queries):
  es = jax.ShapeDtypeStruct((N_EDGES,), jnp.float32)
  return pl.pallas_call(
      _k1_body,
      grid=(NBLK,),
      in_specs=[pl.BlockSpec((2, BK), lambda i: (0, i)),
                pl.BlockSpec((BK, HIDDEN), lambda i: (i, 0)),
                pl.BlockSpec((BK, HIDDEN), lambda i: (i, 0))],
      out_specs=[pl.BlockSpec((BK,), lambda i: (i,)),
                 pl.BlockSpec((BK,), lambda i: (i,)),
                 pl.BlockSpec((BK,), lambda i: (i,)),
                 pl.BlockSpec((BK,), lambda i: (i,)),
                 pl.BlockSpec((BK,), lambda i: (i,)),
                 pl.BlockSpec((8, 128), lambda i: (0, 0))],
      out_shape=[jax.ShapeDtypeStruct((N_EDGES,), jnp.int32),
                 es, es, es, es,
                 jax.ShapeDtypeStruct((8, 128), jnp.float32)],
  )(edge_index, keys, queries)


def _k235_body(s0, s1, s2, s3, dst_hbm, gm_hbm, v_hbm,
               dpart_hbm, opart_hbm,
               dst_v, w0_v, w1_v, w2_v, w3_v, gm_v,
               idxd0, idxd1, idxd2, idxd3,
               idx_a, idx_b, v_a, v_b,
               in_a, in_b, sc_a, sc_b, den_sem,
               den_sh, out_sh):
  c = lax.axis_index("c")
  s = lax.axis_index("s")
  wid = c * 16 + s
  wheads = (w0_v, w1_v, w2_v, w3_v)
  sheads = (s0, s1, s2, s3)

  def zw(j, _):
    w0_v[pl.ds(j * 16, 16)] = jnp.zeros((16,), jnp.float32)
    return 0
  lax.fori_loop(0, SUP // 16, zw, 0)
  pltpu.sync_copy(w0_v.at[pl.ds(0, 1280)],
                  den_sh.at[pl.ds(s * DEN_PER_TILE, 1280)])
  pltpu.sync_copy(w0_v.at[pl.ds(0, 1280)],
                  den_sh.at[pl.ds(s * DEN_PER_TILE + 1280, 1280)])

  def zv(r, _):
    for cc in range(HIDDEN // 16):
      v_a[r, pl.ds(cc * 16, 16)] = jnp.zeros((16,), jnp.float32)
    return 0
  lax.fori_loop(0, CB, zv, 0)
  for zi in range(OUT_RPT // CB):
    pltpu.sync_copy(v_a, out_sh.at[pl.ds(s * OUT_RPT + zi * CB, CB)])
  pltpu.sync_copy(gm_hbm.at[pl.ds(0, 512)], gm_v)
  plsc.subcore_barrier()

  def _mul(v_ref, idx_ref, sbase):
    for j in range(CB // 16):
      idx_ref[pl.ds(j * 16, 16)] = dst_v[pl.ds(sbase + j * 16, 16)]

    def group(g, _):
      wv = [wheads[h][pl.ds(sbase + g * 16, 16)] for h in range(NHEADS)]
      for f in range(16):
        row = g * 16 + f
        for h in range(NHEADS):
          spl = jnp.broadcast_to(wv[h][f:f + 1], (16,))
          for j2 in range(2):
            col = h * 2 * 16 + j2 * 16
            v_ref[row, pl.ds(col, 16)] = v_ref[row, pl.ds(col, 16)] * spl
      return 0
    lax.fori_loop(0, GPC, group, 0)

  idxds = (idxd0, idxd1, idxd2, idxd3)

  def _drain_den():
    for h in range(NHEADS):
      pltpu.make_async_copy(wheads[h], den_sh.at[idxds[h]], den_sem).wait()

  def sup_chunk(si, _):
    base_e = pl.multiple_of(wid * E_PER_W + si * SUP, 8)

    @pl.when(si > 0)
    def _():
      _drain_den()

    pltpu.async_copy(dst_hbm.at[pl.ds(base_e, SUP)], dst_v, sc_b)
    for h in range(NHEADS):
      pltpu.async_copy(sheads[h].at[pl.ds(base_e, SUP)], wheads[h], sc_a)
    pltpu.async_copy(v_hbm.at[pl.ds(base_e, CB)], v_a, in_a)
    pltpu.async_copy(v_hbm.at[pl.ds(base_e + CB, CB)], v_b, in_b)
    pltpu.make_async_copy(dst_hbm.at[pl.ds(base_e, SUP)], dst_v, sc_b).wait()
    for h in range(NHEADS):
      pltpu.make_async_copy(sheads[h].at[pl.ds(base_e, SUP)],
                            wheads[h], sc_a).wait()

    for h in range(NHEADS):
      gh = gm_v[pl.ds(h * 128, 16)]
      wh = wheads[h]
      ih = idxds[h]

      def vbody(j, _):
        off = j * 16
        wh[pl.ds(off, 16)] = jnp.exp(wh[pl.ds(off, 16)] - gh)
        dv = dst_v[pl.ds(off, 16)]
        ih[pl.ds(off, 16)] = dv + h * N_PAD
        return 0
      lax.fori_loop(0, VPC, vbody, 0)
      pltpu.async_copy(wh, den_sh.at[ih], den_sem, add=True)


    def pair(p, _):
      sub_a = 2 * p
      sub_b = 2 * p + 1
      pltpu.make_async_copy(v_hbm.at[pl.ds(base_e, CB)], v_a, in_a).wait()
      _mul(v_a, idx_a, sub_a * CB)
      pltpu.async_copy(v_a, out_sh.at[idx_a], sc_a, add=True)
      pltpu.make_async_copy(v_hbm.at[pl.ds(base_e, CB)], v_b, in_b).wait()
      _mul(v_b, idx_b, sub_b * CB)
      pltpu.async_copy(v_b, out_sh.at[idx_b], sc_b, add=True)
      pltpu.make_async_copy(v_a, out_sh.at[idx_a], sc_a).wait()
      pltpu.async_copy(v_hbm.at[pl.ds(base_e + (sub_a + 2) * CB, CB)],
                       v_a, in_a)
      pltpu.make_async_copy(v_b, out_sh.at[idx_b], sc_b).wait()

      @pl.when(p < N_CB // 2 - 1)
      def _():
        pltpu.async_copy(v_hbm.at[pl.ds(base_e + (sub_b + 2) * CB, CB)],
                         v_b, in_b)
      return 0
    lax.fori_loop(0, N_CB // 2, pair, 0)
    pltpu.make_async_copy(v_hbm.at[pl.ds(base_e, CB)], v_a, in_a).wait()
    _mul(v_a, idx_a, (N_CB - 1) * CB)
    pltpu.async_copy(v_a, out_sh.at[idx_a], sc_a, add=True)
    pltpu.make_async_copy(v_a, out_sh.at[idx_a], sc_a).wait()
    return 0
  lax.fori_loop(0, N_SUP, sup_chunk, 0)
  _drain_den()

  plsc.subcore_barrier()
  drow = c * NHEADS + s // NHEADS
  dcol = pl.multiple_of((s % NHEADS) * DEN_PER_TILE, 8)
  pltpu.sync_copy(den_sh.at[pl.ds(s * DEN_PER_TILE, DEN_PER_TILE)],
                  dpart_hbm.at[drow].at[pl.ds(dcol, DEN_PER_TILE)])
  pltpu.sync_copy(out_sh.at[pl.ds(s * OUT_RPT, OUT_RPT)],
                  opart_hbm.at[c].at[pl.ds(s * OUT_RPT, OUT_RPT)])


def _k235(s0, s1, s2, s3, dst, gmaxflat, values):
  mesh = plsc.VectorSubcoreMesh(core_axis_name="c", subcore_axis_name="s")
  return pl.kernel(
      _k235_body,
      out_type=[jax.ShapeDtypeStruct((2 * NHEADS, N_PAD), jnp.float32),
                jax.ShapeDtypeStruct((2, N_PAD, HIDDEN), jnp.float32)],
      mesh=mesh,
      scratch_types=[pltpu.VMEM((SUP,), jnp.int32),
                     pltpu.VMEM((SUP,), jnp.float32),
                     pltpu.VMEM((SUP,), jnp.float32),
                     pltpu.VMEM((SUP,), jnp.float32),
                     pltpu.VMEM((SUP,), jnp.float32),
                     pltpu.VMEM((512,), jnp.float32),
                     pltpu.VMEM((SUP,), jnp.int32),
                     pltpu.VMEM((SUP,), jnp.int32),
                     pltpu.VMEM((SUP,), jnp.int32),
                     pltpu.VMEM((SUP,), jnp.int32),
                     pltpu.VMEM((CB,), jnp.int32),
                     pltpu.VMEM((CB,), jnp.int32),
                     pltpu.VMEM((CB, HIDDEN), jnp.float32),
                     pltpu.VMEM((CB, HIDDEN), jnp.float32),
                     pltpu.SemaphoreType.DMA,
                     pltpu.SemaphoreType.DMA,
                     pltpu.SemaphoreType.DMA,
                     pltpu.SemaphoreType.DMA,
                     pltpu.SemaphoreType.DMA,
                     pltpu.VMEM_SHARED((DEN_PAD,), jnp.float32),
                     pltpu.VMEM_SHARED((N_PAD, HIDDEN), jnp.float32)],
  )(s0, s1, s2, s3, dst, gmaxflat, values)


BN = 2048


def _k6_body(p_ref, d_ref, o_ref):
  dsum = d_ref[0:NHEADS, :] + d_ref[NHEADS:2 * NHEADS, :]
  dexp = jax.lax.dot_general(dsum, _sel4(),
                             (((0,), (0,)), ((), ())),
                             preferred_element_type=jnp.float32)
  o = p_ref[0] + p_ref[1]
  o_ref[...] = jnp.where(dexp > 0, o / dexp, 0.0)


def _k6(opart, dpart8):
  return pl.pallas_call(
      _k6_body,
      grid=(pl.cdiv(N_NODES, BN),),
      in_specs=[pl.BlockSpec((2, BN, HIDDEN), lambda i: (0, i, 0)),
                pl.BlockSpec((2 * NHEADS, BN), lambda i: (0, i))],
      out_specs=pl.BlockSpec((BN, HIDDEN), lambda i: (i, 0)),
      out_shape=jax.ShapeDtypeStruct((N_NODES, HIDDEN), jnp.float32),
  )(opart, dpart8)


@jax.jit
def kernel(edge_index, keys, queries, values):
  dst, s0, s1, s2, s3, gmax8 = _k1(edge_index, keys, queries)
  dpart, opart = _k235(s0, s1, s2, s3, dst, gmax8.reshape(-1), values)
  return _k6(opart, dpart)
